# Initial kernel scaffold; baseline (speedup 1.0000x reference)
import jax
import jax.numpy as jnp
from jax.experimental import pallas as pl


def kernel(encoder_input_ids, encoder_attention_mask, embed_table):
    raise NotImplementedError("write your pallas kernel here")


# speedup vs baseline: 1.5206x; 1.5206x over previous
import functools

import jax
import jax.numpy as jnp
from jax import lax
from jax.experimental import pallas as pl
from jax.experimental.pallas import tpu as pltpu
from jax.experimental.pallas import tpu_sc as plsc

VOCAB = 32128
D_MODEL = 1024
BATCH = 4
SEQ = 2048

NC, NS, L = 2, 16, 16
NW = NC * NS
TOTAL = BATCH * SEQ
B_PER_W = TOTAL // NW
CHUNK = 32
NCH = B_PER_W // CHUNK

_F32_MIN = float(jnp.finfo(jnp.float32).min)


def _sc_body(ids_hbm, mask_hbm, table_hbm, out_hbm, ext_hbm,
             idx_v, buf0, buf1, mask_v, ext_v, sem0, sem1):
    wid = lax.axis_index("s") * NC + lax.axis_index("c")
    base = wid * B_PER_W

    pltpu.sync_copy(ids_hbm.at[wid], idx_v)

    bufs = (buf0, buf1)
    sems = (sem0, sem1)

    copies = [pltpu.async_copy(table_hbm.at[idx_v.at[0]], buf0, sem0)]

    pltpu.sync_copy(mask_hbm.at[pl.ds(base, B_PER_W)], mask_v)
    for m in range(B_PER_W // L):
        v = mask_v[pl.ds(m * L, L)].astype(jnp.float32)
        ext_v[pl.ds(m * L, L)] = (1.0 - v) * _F32_MIN
    pltpu.sync_copy(ext_v, ext_hbm.at[pl.ds(base, B_PER_W)])

    for j in range(NCH):
        if j + 1 < NCH:
            copies.append(pltpu.async_copy(
                table_hbm.at[idx_v.at[j + 1]], bufs[(j + 1) % 2],
                sems[(j + 1) % 2]))
        copies[j].wait()
        pltpu.sync_copy(bufs[j % 2], out_hbm.at[pl.ds(base + j * CHUNK, CHUNK)])


@functools.partial(jax.jit, static_argnames=())
def _sc_embed(ids2d, mask_flat, table):
    mesh = plsc.VectorSubcoreMesh(core_axis_name="c", subcore_axis_name="s")
    fn = pl.kernel(
        _sc_body,
        out_type=[
            jax.ShapeDtypeStruct((TOTAL, D_MODEL), jnp.float32),
            jax.ShapeDtypeStruct((TOTAL,), jnp.float32),
        ],
        mesh=mesh,
        scratch_types=[
            pltpu.VMEM((NCH, CHUNK), jnp.int32),
            pltpu.VMEM((CHUNK, D_MODEL), jnp.float32),
            pltpu.VMEM((CHUNK, D_MODEL), jnp.float32),
            pltpu.VMEM((B_PER_W,), jnp.int32),
            pltpu.VMEM((B_PER_W,), jnp.float32),
            pltpu.SemaphoreType.DMA,
            pltpu.SemaphoreType.DMA,
        ],
        name="t5_embed_gather_sc",
    )
    return fn(ids2d, mask_flat, table)


def kernel(encoder_input_ids, encoder_attention_mask, embed_table):
    ids2d = encoder_input_ids.astype(jnp.int32).reshape(NW, NCH, CHUNK)
    mask_flat = encoder_attention_mask.astype(jnp.int32).reshape(TOTAL)
    hidden_flat, ext_flat = _sc_embed(ids2d, mask_flat, embed_table)
    hidden = hidden_flat.reshape(BATCH, SEQ, D_MODEL)
    ext = ext_flat.reshape(BATCH, 1, 1, SEQ)
    return (encoder_attention_mask, ext, hidden)


# speedup vs baseline: 1.5648x; 1.0291x over previous
import functools

import jax
import jax.numpy as jnp
from jax import lax
from jax.experimental import pallas as pl
from jax.experimental.pallas import tpu as pltpu
from jax.experimental.pallas import tpu_sc as plsc

VOCAB = 32128
D_MODEL = 1024
BATCH = 4
SEQ = 2048

NC, NS, L = 2, 16, 16
NW = NC * NS
TOTAL = BATCH * SEQ
B_PER_W = TOTAL // NW
CHUNK = 32
NCH = B_PER_W // CHUNK
NBUF = 3

_F32_MIN = float(jnp.finfo(jnp.float32).min)


def _sc_body(ids_hbm, mask_hbm, table_hbm, out_hbm, ext_hbm,
             idx_v, buf0, buf1, buf2, mask_v, ext_v,
             gsem0, gsem1, gsem2, wsem0, wsem1, wsem2):
    wid = lax.axis_index("s") * NC + lax.axis_index("c")
    base = wid * B_PER_W

    pltpu.sync_copy(ids_hbm.at[wid], idx_v)

    bufs = (buf0, buf1, buf2)
    gsems = (gsem0, gsem1, gsem2)
    wsems = (wsem0, wsem1, wsem2)

    def gather(j):
        return pltpu.async_copy(
            table_hbm.at[idx_v.at[j]], bufs[j % NBUF], gsems[j % NBUF])

    gcopies = {j: gather(j) for j in range(NBUF - 1)}

    pltpu.sync_copy(mask_hbm.at[pl.ds(base, B_PER_W)], mask_v)
    for m in range(B_PER_W // L):
        v = mask_v[pl.ds(m * L, L)].astype(jnp.float32)
        ext_v[pl.ds(m * L, L)] = (1.0 - v) * _F32_MIN
    pltpu.sync_copy(ext_v, ext_hbm.at[pl.ds(base, B_PER_W)])

    wcopies = {}
    for j in range(NCH):
        if j >= 1:
            wcopies[j - 1].wait()
        if j + NBUF - 1 < NCH:
            gcopies[j + NBUF - 1] = gather(j + NBUF - 1)
        gcopies[j].wait()
        wcopies[j] = pltpu.async_copy(
            bufs[j % NBUF], out_hbm.at[pl.ds(base + j * CHUNK, CHUNK)],
            wsems[j % NBUF])
    wcopies[NCH - 1].wait()


@functools.partial(jax.jit, static_argnames=())
def _sc_embed(ids2d, mask_flat, table):
    mesh = plsc.VectorSubcoreMesh(core_axis_name="c", subcore_axis_name="s")
    fn = pl.kernel(
        _sc_body,
        out_type=[
            jax.ShapeDtypeStruct((TOTAL, D_MODEL), jnp.float32),
            jax.ShapeDtypeStruct((TOTAL,), jnp.float32),
        ],
        mesh=mesh,
        scratch_types=[
            pltpu.VMEM((NCH, CHUNK), jnp.int32),
            pltpu.VMEM((CHUNK, D_MODEL), jnp.float32),
            pltpu.VMEM((CHUNK, D_MODEL), jnp.float32),
            pltpu.VMEM((CHUNK, D_MODEL), jnp.float32),
            pltpu.VMEM((B_PER_W,), jnp.int32),
            pltpu.VMEM((B_PER_W,), jnp.float32),
            pltpu.SemaphoreType.DMA,
            pltpu.SemaphoreType.DMA,
            pltpu.SemaphoreType.DMA,
            pltpu.SemaphoreType.DMA,
            pltpu.SemaphoreType.DMA,
            pltpu.SemaphoreType.DMA,
        ],
        name="t5_embed_gather_sc",
    )
    return fn(ids2d, mask_flat, table)


def kernel(encoder_input_ids, encoder_attention_mask, embed_table):
    ids2d = encoder_input_ids.astype(jnp.int32).reshape(NW, NCH, CHUNK)
    mask_flat = encoder_attention_mask.astype(jnp.int32).reshape(TOTAL)
    hidden_flat, ext_flat = _sc_embed(ids2d, mask_flat, embed_table)
    hidden = hidden_flat.reshape(BATCH, SEQ, D_MODEL)
    ext = ext_flat.reshape(BATCH, 1, 1, SEQ)
    return (encoder_attention_mask, ext, hidden)
